# baseline (speedup 1.0000x reference)
import functools

import jax
import jax.numpy as jnp
from jax import lax
from jax.experimental import pallas as pl
from jax.experimental.pallas import tpu as pltpu
from jax.experimental.pallas import tpu_sc as plsc

N_EXP = 8
TOPK = 2
H = 2048
I = 768
T = 8192
TM = 256
G = T * TOPK // TM + N_EXP
P = G * TM
TT = 256


def _cumsum0(x):
    n = x.shape[0]
    d = 1
    while d < n:
        x = x + jnp.concatenate(
            [jnp.zeros((d, x.shape[1]), x.dtype), x[:-d]], axis=0)
        d *= 2
    return x


def _routing_body(lg_ref, p0_ref, p1_ref, wa_ref, wb_ref, cnt_ref):
    lg = lg_ref[...]
    idx = jax.lax.broadcasted_iota(jnp.int32, (T, N_EXP), 1)
    m1 = jnp.max(lg, axis=1, keepdims=True)
    a1 = jnp.min(jnp.where(lg == m1, idx, N_EXP), axis=1, keepdims=True)
    masked = jnp.where(idx == a1, -jnp.inf, lg)
    m2 = jnp.max(masked, axis=1, keepdims=True)
    a2 = jnp.min(jnp.where(masked == m2, idx, N_EXP), axis=1, keepdims=True)
    w0 = jax.lax.logistic(m1 - m2)
    wa_ref[...] = jnp.broadcast_to(w0, (T, N_EXP))
    wb_ref[...] = jnp.broadcast_to(1.0 - w0, (T, N_EXP))
    oh0 = (idx == a1).astype(jnp.int32)
    oh1 = (idx == a2).astype(jnp.int32)
    c0 = _cumsum0(oh0)
    c1 = _cumsum0(oh1)
    cnt0 = c0[T - 1:T, :]
    cnt = cnt0 + c1[T - 1:T, :]
    cnt_ref[...] = cnt
    pg = ((cnt + TM - 1) // TM) * TM
    starts = pg
    d = 1
    while d < N_EXP:
        starts = starts + jnp.concatenate(
            [jnp.zeros((1, d), starts.dtype), starts[:, :-d]], axis=1)
        d *= 2
    starts = starts - pg
    p0 = jnp.sum(oh0 * (starts + c0 - 1), axis=1, keepdims=True)
    p1 = jnp.sum(oh1 * (starts + cnt0 + c1 - 1), axis=1, keepdims=True)
    p0_ref[...] = p0
    p1_ref[...] = p1


def _routing(router_logits):
    p0, p1, wa, wb, cnt = pl.pallas_call(
        _routing_body,
        out_shape=[
            jax.ShapeDtypeStruct((T, 1), jnp.int32),
            jax.ShapeDtypeStruct((T, 1), jnp.int32),
            jax.ShapeDtypeStruct((T, N_EXP), jnp.float32),
            jax.ShapeDtypeStruct((T, N_EXP), jnp.float32),
            jax.ShapeDtypeStruct((1, N_EXP), jnp.int32),
        ],
    )(router_logits)
    cnt = cnt[0]
    pg = ((cnt + TM - 1) // TM) * TM
    starts = jnp.cumsum(pg) - pg
    tile_start = jnp.arange(G, dtype=jnp.int32) * TM
    eid = jnp.clip(
        jnp.searchsorted(starts.astype(jnp.int32), tile_start, side="right")
        - 1, 0, N_EXP - 1).astype(jnp.int32)
    active = (tile_start < (starts[eid] + cnt[eid])).astype(jnp.int32)
    return wa, wb, p0.reshape(T), p1.reshape(T), eid, active


NI = 2
IB = I // NI


def _ffn_body(eid_ref, act_ref, x_ref, wg_ref, wu_ref, wd_ref, o_ref,
              xbf, wgs, wus, wds):
    i = pl.program_id(0)
    n = pl.program_id(1)
    changed = jnp.logical_or(
        i == 0, eid_ref[i] != eid_ref[jnp.maximum(i - 1, 0)])
    act = act_ref[i] == 1

    @pl.when(jnp.logical_and(act, n == 0))
    def _():
        xbf[...] = x_ref[...].astype(jnp.bfloat16)

    @pl.when(jnp.logical_and(act, changed))
    def _():
        wgs[:, pl.ds(n * IB, IB)] = wg_ref[0].astype(jnp.bfloat16)
        wus[:, pl.ds(n * IB, IB)] = wu_ref[0].astype(jnp.bfloat16)
        wds[pl.ds(n * IB, IB), :] = wd_ref[0].astype(jnp.bfloat16)

    @pl.when(act)
    def _():
        x = xbf[...]
        g = jnp.dot(x, wgs[:, pl.ds(n * IB, IB)],
                    preferred_element_type=jnp.float32)
        u = jnp.dot(x, wus[:, pl.ds(n * IB, IB)],
                    preferred_element_type=jnp.float32)
        a = (g * jax.lax.logistic(g) * u).astype(jnp.bfloat16)
        part = jnp.dot(a, wds[pl.ds(n * IB, IB), :],
                       preferred_element_type=jnp.float32)

        @pl.when(n == 0)
        def _():
            o_ref[...] = part

        @pl.when(n != 0)
        def _():
            o_ref[...] = o_ref[...] + part


def _ffn(eid, active, xs, wg, wu, wd):
    grid_spec = pltpu.PrefetchScalarGridSpec(
        num_scalar_prefetch=2,
        grid=(G, NI),
        in_specs=[
            pl.BlockSpec((TM, H), lambda i, n, eid, act: (i, 0)),
            pl.BlockSpec((1, H, IB), lambda i, n, eid, act: (eid[i], 0, n)),
            pl.BlockSpec((1, H, IB), lambda i, n, eid, act: (eid[i], 0, n)),
            pl.BlockSpec((1, IB, H), lambda i, n, eid, act: (eid[i], n, 0)),
        ],
        out_specs=pl.BlockSpec((TM, H), lambda i, n, eid, act: (i, 0)),
        scratch_shapes=[
            pltpu.VMEM((TM, H), jnp.bfloat16),
            pltpu.VMEM((H, I), jnp.bfloat16),
            pltpu.VMEM((H, I), jnp.bfloat16),
            pltpu.VMEM((I, H), jnp.bfloat16),
        ],
    )
    return pl.pallas_call(
        _ffn_body,
        grid_spec=grid_spec,
        out_shape=jax.ShapeDtypeStruct((P, H), jnp.float32),
    )(eid, active, xs, wg, wu, wd)


def _combine_body(c0_ref, c1_ref, wa_ref, wb_ref, o_ref):
    o_ref[...] = (c0_ref[...] * wa_ref[:, 0:1]
                  + c1_ref[...] * wb_ref[:, 0:1])


def _combine(c, wa, wb):
    return pl.pallas_call(
        _combine_body,
        grid=(T // TT,),
        in_specs=[
            pl.BlockSpec((TT, H), lambda i: (i, 0)),
            pl.BlockSpec((TT, H), lambda i: (i + T // TT, 0)),
            pl.BlockSpec((TT, N_EXP), lambda i: (i, 0)),
            pl.BlockSpec((TT, N_EXP), lambda i: (i, 0)),
        ],
        out_specs=pl.BlockSpec((TT, H), lambda i: (i, 0)),
        out_shape=jax.ShapeDtypeStruct((T, H), jnp.float32),
    )(c, c, wa, wb)


NC = 2
NS = 16
NW = NC * NS
SL = H // 128

def _sc_mesh():
    return plsc.VectorSubcoreMesh(core_axis_name="c", subcore_axis_name="s")

RB = 16
_DISP_NIT = T // NW // RB
_COMB_NIT = TOPK * T // NW // RB


def _sc_dispatch(hid3, p0_3, p1_3):
    @functools.partial(
        pl.kernel,
        out_type=jax.ShapeDtypeStruct((P, H), jnp.float32),
        mesh=_sc_mesh(),
        scratch_types=[
            pltpu.VMEM((RB, H), jnp.float32),
            pltpu.VMEM((RB, H), jnp.float32),
            pltpu.VMEM((_DISP_NIT, RB), jnp.int32),
            pltpu.VMEM((_DISP_NIT, RB), jnp.int32),
            pltpu.SemaphoreType.DMA,
            pltpu.SemaphoreType.DMA,
            pltpu.SemaphoreType.DMA,
            pltpu.SemaphoreType.DMA,
        ],
    )
    def k(hid_hbm, p0_hbm, p1_hbm, xs_hbm,
          rows0, rows1, i0_v, i1_v, ls0, ls1, ss0, ss1):
        wid = lax.axis_index("s") * NC + lax.axis_index("c")
        base = wid * (T // NW)
        rows = (rows0, rows1)
        lsem = (ls0, ls1)
        ssem = (ss0, ss1)
        pltpu.sync_copy(p0_hbm.at[wid], i0_v)
        pltpu.sync_copy(p1_hbm.at[wid], i1_v)
        ld = [pltpu.async_copy(hid_hbm.at[pl.ds(base, RB)], rows0, ls0), None]
        st = [[], []]
        for j in range(_DISP_NIT):
            b = j & 1
            ld[b].wait()
            if j + 1 < _DISP_NIT:
                for cp in st[1 - b]:
                    cp.wait()
                ld[1 - b] = pltpu.async_copy(
                    hid_hbm.at[pl.ds(base + (j + 1) * RB, RB)],
                    rows[1 - b], lsem[1 - b])
            st[b] = [
                pltpu.async_copy(rows[b], xs_hbm.at[i0_v[j, :]], ssem[b]),
                pltpu.async_copy(rows[b], xs_hbm.at[i1_v[j, :]], ssem[b]),
            ]
        for cp in st[0] + st[1]:
            cp.wait()

    return k(hid3, p0_3, p1_3)


def _sc_combine_gather(d3, p_all_3):
    @functools.partial(
        pl.kernel,
        out_type=jax.ShapeDtypeStruct((TOPK * T, H), jnp.float32),
        mesh=_sc_mesh(),
        scratch_types=[
            pltpu.VMEM((RB, H), jnp.float32),
            pltpu.VMEM((RB, H), jnp.float32),
            pltpu.VMEM((_COMB_NIT, RB), jnp.int32),
            pltpu.SemaphoreType.DMA,
            pltpu.SemaphoreType.DMA,
            pltpu.SemaphoreType.DMA,
            pltpu.SemaphoreType.DMA,
        ],
    )
    def k(d_hbm, p_hbm, c_hbm, rows0, rows1, pidx_v, gs0, gs1, ws0, ws1):
        wid = lax.axis_index("s") * NC + lax.axis_index("c")
        base = wid * (TOPK * T // NW)
        rows = (rows0, rows1)
        gsem = (gs0, gs1)
        wsem = (ws0, ws1)
        pltpu.sync_copy(p_hbm.at[wid], pidx_v)
        g = [pltpu.async_copy(d_hbm.at[pidx_v[0, :]], rows0, gs0), None]
        wr = [None, None]
        for j in range(_COMB_NIT):
            b = j & 1
            g[b].wait()
            if j + 1 < _COMB_NIT:
                if wr[1 - b] is not None:
                    wr[1 - b].wait()
                g[1 - b] = pltpu.async_copy(
                    d_hbm.at[pidx_v[j + 1, :]], rows[1 - b], gsem[1 - b])
            wr[b] = pltpu.async_copy(
                rows[b], c_hbm.at[pl.ds(base + j * RB, RB)], wsem[b])
        wr[0].wait()
        wr[1].wait()

    return k(d3, p_all_3)


def kernel(hidden_states, router_logits, gate_proj, up_proj, down_proj):
    wa, wb, p0, p1, eid, active = _routing(router_logits)
    wg, wu, wd = gate_proj, up_proj, down_proj
    xs = _sc_dispatch(hidden_states,
                      p0.reshape(NW, _DISP_NIT, RB),
                      p1.reshape(NW, _DISP_NIT, RB))
    d = _ffn(eid, active, xs, wg, wu, wd)
    p_all = jnp.concatenate([p0, p1]).reshape(NW, _COMB_NIT, RB)
    c = _sc_combine_gather(d, p_all)
    return _combine(c, wa, wb)


# speedup vs baseline: 1.3536x; 1.3536x over previous
import functools

import jax
import jax.numpy as jnp
from jax import lax
from jax.experimental import pallas as pl
from jax.experimental.pallas import tpu as pltpu
from jax.experimental.pallas import tpu_sc as plsc

N_EXP = 8
TOPK = 2
H = 2048
I = 768
T = 8192
TM = 512
G = T * TOPK // TM + N_EXP
P = G * TM
TT = 256


def _cumsum0(x):
    n = x.shape[0]
    d = 1
    while d < n:
        x = x + jnp.concatenate(
            [jnp.zeros((d, x.shape[1]), x.dtype), x[:-d]], axis=0)
        d *= 2
    return x


def _routing_body(lg_ref, p0_ref, p1_ref, wa_ref, wb_ref, cnt_ref):
    lg = lg_ref[...]
    idx = jax.lax.broadcasted_iota(jnp.int32, (T, N_EXP), 1)
    m1 = jnp.max(lg, axis=1, keepdims=True)
    a1 = jnp.min(jnp.where(lg == m1, idx, N_EXP), axis=1, keepdims=True)
    masked = jnp.where(idx == a1, -jnp.inf, lg)
    m2 = jnp.max(masked, axis=1, keepdims=True)
    a2 = jnp.min(jnp.where(masked == m2, idx, N_EXP), axis=1, keepdims=True)
    w0 = jax.lax.logistic(m1 - m2)
    wa_ref[...] = jnp.broadcast_to(w0, (T, N_EXP))
    wb_ref[...] = jnp.broadcast_to(1.0 - w0, (T, N_EXP))
    oh0 = (idx == a1).astype(jnp.int32)
    oh1 = (idx == a2).astype(jnp.int32)
    c0 = _cumsum0(oh0)
    c1 = _cumsum0(oh1)
    cnt0 = c0[T - 1:T, :]
    cnt = cnt0 + c1[T - 1:T, :]
    cnt_ref[...] = cnt
    pg = ((cnt + TM - 1) // TM) * TM
    starts = pg
    d = 1
    while d < N_EXP:
        starts = starts + jnp.concatenate(
            [jnp.zeros((1, d), starts.dtype), starts[:, :-d]], axis=1)
        d *= 2
    starts = starts - pg
    p0 = jnp.sum(oh0 * (starts + c0 - 1), axis=1, keepdims=True)
    p1 = jnp.sum(oh1 * (starts + cnt0 + c1 - 1), axis=1, keepdims=True)
    p0_ref[...] = p0
    p1_ref[...] = p1


def _routing(router_logits):
    p0, p1, wa, wb, cnt = pl.pallas_call(
        _routing_body,
        out_shape=[
            jax.ShapeDtypeStruct((T, 1), jnp.int32),
            jax.ShapeDtypeStruct((T, 1), jnp.int32),
            jax.ShapeDtypeStruct((T, N_EXP), jnp.float32),
            jax.ShapeDtypeStruct((T, N_EXP), jnp.float32),
            jax.ShapeDtypeStruct((1, N_EXP), jnp.int32),
        ],
    )(router_logits)
    cnt = cnt[0]
    pg = ((cnt + TM - 1) // TM) * TM
    starts = jnp.cumsum(pg) - pg
    tile_start = jnp.arange(G, dtype=jnp.int32) * TM
    eid = jnp.clip(
        jnp.searchsorted(starts.astype(jnp.int32), tile_start, side="right")
        - 1, 0, N_EXP - 1).astype(jnp.int32)
    active = (tile_start < (starts[eid] + cnt[eid])).astype(jnp.int32)
    return wa, wb, p0.reshape(T), p1.reshape(T), eid, active


def _ffn_body(eid_ref, act_ref, x_ref, wg_ref, wu_ref, wd_ref, o_ref):
    i = pl.program_id(0)

    @pl.when(act_ref[i] == 1)
    def _():
        x = x_ref[...].astype(jnp.bfloat16)
        g = jnp.dot(x, wg_ref[0], preferred_element_type=jnp.float32)
        u = jnp.dot(x, wu_ref[0], preferred_element_type=jnp.float32)
        a = (g * jax.lax.logistic(g) * u).astype(jnp.bfloat16)
        o_ref[...] = jnp.dot(a, wd_ref[0], preferred_element_type=jnp.float32)


def _ffn(eid, active, xs, wg, wu, wd):
    grid_spec = pltpu.PrefetchScalarGridSpec(
        num_scalar_prefetch=2,
        grid=(G,),
        in_specs=[
            pl.BlockSpec((TM, H), lambda i, eid, act: (i, 0)),
            pl.BlockSpec((1, H, I), lambda i, eid, act: (eid[i], 0, 0)),
            pl.BlockSpec((1, H, I), lambda i, eid, act: (eid[i], 0, 0)),
            pl.BlockSpec((1, I, H), lambda i, eid, act: (eid[i], 0, 0)),
        ],
        out_specs=pl.BlockSpec((TM, H), lambda i, eid, act: (i, 0)),
    )
    return pl.pallas_call(
        _ffn_body,
        grid_spec=grid_spec,
        out_shape=jax.ShapeDtypeStruct((P, H), jnp.float32),
    )(eid, active, xs, wg, wu, wd)


def _combine_body(c0_ref, c1_ref, wa_ref, wb_ref, o_ref):
    o_ref[...] = (c0_ref[...] * wa_ref[:, 0:1]
                  + c1_ref[...] * wb_ref[:, 0:1])


def _combine(c, wa, wb):
    return pl.pallas_call(
        _combine_body,
        grid=(T // TT,),
        in_specs=[
            pl.BlockSpec((TT, H), lambda i: (i, 0)),
            pl.BlockSpec((TT, H), lambda i: (i + T // TT, 0)),
            pl.BlockSpec((TT, N_EXP), lambda i: (i, 0)),
            pl.BlockSpec((TT, N_EXP), lambda i: (i, 0)),
        ],
        out_specs=pl.BlockSpec((TT, H), lambda i: (i, 0)),
        out_shape=jax.ShapeDtypeStruct((T, H), jnp.float32),
    )(c, c, wa, wb)


NC = 2
NS = 16
NW = NC * NS
SL = H // 128

def _sc_mesh():
    return plsc.VectorSubcoreMesh(core_axis_name="c", subcore_axis_name="s")

RB = 16
_DISP_NIT = T // NW // RB
_COMB_NIT = TOPK * T // NW // RB


def _sc_dispatch(hid3, p0_3, p1_3):
    @functools.partial(
        pl.kernel,
        out_type=jax.ShapeDtypeStruct((P, H), jnp.float32),
        mesh=_sc_mesh(),
        scratch_types=[
            pltpu.VMEM((RB, H), jnp.float32),
            pltpu.VMEM((RB, H), jnp.float32),
            pltpu.VMEM((_DISP_NIT, RB), jnp.int32),
            pltpu.VMEM((_DISP_NIT, RB), jnp.int32),
            pltpu.SemaphoreType.DMA,
            pltpu.SemaphoreType.DMA,
            pltpu.SemaphoreType.DMA,
            pltpu.SemaphoreType.DMA,
        ],
    )
    def k(hid_hbm, p0_hbm, p1_hbm, xs_hbm,
          rows0, rows1, i0_v, i1_v, ls0, ls1, ss0, ss1):
        wid = lax.axis_index("s") * NC + lax.axis_index("c")
        base = wid * (T // NW)
        rows = (rows0, rows1)
        lsem = (ls0, ls1)
        ssem = (ss0, ss1)
        pltpu.sync_copy(p0_hbm.at[wid], i0_v)
        pltpu.sync_copy(p1_hbm.at[wid], i1_v)
        ld = [pltpu.async_copy(hid_hbm.at[pl.ds(base, RB)], rows0, ls0), None]
        st = [[], []]
        for j in range(_DISP_NIT):
            b = j & 1
            ld[b].wait()
            if j + 1 < _DISP_NIT:
                for cp in st[1 - b]:
                    cp.wait()
                ld[1 - b] = pltpu.async_copy(
                    hid_hbm.at[pl.ds(base + (j + 1) * RB, RB)],
                    rows[1 - b], lsem[1 - b])
            st[b] = [
                pltpu.async_copy(rows[b], xs_hbm.at[i0_v[j, :]], ssem[b]),
                pltpu.async_copy(rows[b], xs_hbm.at[i1_v[j, :]], ssem[b]),
            ]
        for cp in st[0] + st[1]:
            cp.wait()

    return k(hid3, p0_3, p1_3)


def _sc_combine_gather(d3, p_all_3):
    @functools.partial(
        pl.kernel,
        out_type=jax.ShapeDtypeStruct((TOPK * T, H), jnp.float32),
        mesh=_sc_mesh(),
        scratch_types=[
            pltpu.VMEM((RB, H), jnp.float32),
            pltpu.VMEM((RB, H), jnp.float32),
            pltpu.VMEM((_COMB_NIT, RB), jnp.int32),
            pltpu.SemaphoreType.DMA,
            pltpu.SemaphoreType.DMA,
            pltpu.SemaphoreType.DMA,
            pltpu.SemaphoreType.DMA,
        ],
    )
    def k(d_hbm, p_hbm, c_hbm, rows0, rows1, pidx_v, gs0, gs1, ws0, ws1):
        wid = lax.axis_index("s") * NC + lax.axis_index("c")
        base = wid * (TOPK * T // NW)
        rows = (rows0, rows1)
        gsem = (gs0, gs1)
        wsem = (ws0, ws1)
        pltpu.sync_copy(p_hbm.at[wid], pidx_v)
        g = [pltpu.async_copy(d_hbm.at[pidx_v[0, :]], rows0, gs0), None]
        wr = [None, None]
        for j in range(_COMB_NIT):
            b = j & 1
            g[b].wait()
            if j + 1 < _COMB_NIT:
                if wr[1 - b] is not None:
                    wr[1 - b].wait()
                g[1 - b] = pltpu.async_copy(
                    d_hbm.at[pidx_v[j + 1, :]], rows[1 - b], gsem[1 - b])
            wr[b] = pltpu.async_copy(
                rows[b], c_hbm.at[pl.ds(base + j * RB, RB)], wsem[b])
        wr[0].wait()
        wr[1].wait()

    return k(d3, p_all_3)


def kernel(hidden_states, router_logits, gate_proj, up_proj, down_proj):
    wa, wb, p0, p1, eid, active = _routing(router_logits)
    wg = gate_proj.astype(jnp.bfloat16)
    wu = up_proj.astype(jnp.bfloat16)
    wd = down_proj.astype(jnp.bfloat16)
    xs = _sc_dispatch(hidden_states,
                      p0.reshape(NW, _DISP_NIT, RB),
                      p1.reshape(NW, _DISP_NIT, RB))
    d = _ffn(eid, active, xs, wg, wu, wd)
    p_all = jnp.concatenate([p0, p1]).reshape(NW, _COMB_NIT, RB)
    c = _sc_combine_gather(d, p_all)
    return _combine(c, wa, wb)


# speedup vs baseline: 1.3597x; 1.0045x over previous
import functools

import jax
import jax.numpy as jnp
from jax import lax
from jax.experimental import pallas as pl
from jax.experimental.pallas import tpu as pltpu
from jax.experimental.pallas import tpu_sc as plsc

N_EXP = 8
TOPK = 2
H = 2048
I = 768
T = 8192
TM = 512
G = T * TOPK // TM + N_EXP
P = G * TM
TT = 512


def _cumsum0(x):
    n = x.shape[0]
    d = 1
    while d < n:
        x = x + jnp.concatenate(
            [jnp.zeros((d, x.shape[1]), x.dtype), x[:-d]], axis=0)
        d *= 2
    return x


def _routing_body(lg_ref, p0_ref, p1_ref, wa_ref, wb_ref, cnt_ref):
    lg = lg_ref[...]
    idx = jax.lax.broadcasted_iota(jnp.int32, (T, N_EXP), 1)
    m1 = jnp.max(lg, axis=1, keepdims=True)
    a1 = jnp.min(jnp.where(lg == m1, idx, N_EXP), axis=1, keepdims=True)
    masked = jnp.where(idx == a1, -jnp.inf, lg)
    m2 = jnp.max(masked, axis=1, keepdims=True)
    a2 = jnp.min(jnp.where(masked == m2, idx, N_EXP), axis=1, keepdims=True)
    w0 = jax.lax.logistic(m1 - m2)
    wa_ref[...] = jnp.broadcast_to(w0, (T, N_EXP))
    wb_ref[...] = jnp.broadcast_to(1.0 - w0, (T, N_EXP))
    oh0 = (idx == a1).astype(jnp.int32)
    oh1 = (idx == a2).astype(jnp.int32)
    enc = _cumsum0(oh0 + (oh1 << 16))
    c0 = enc & 0xFFFF
    c1 = jax.lax.shift_right_logical(enc, 16)
    cnt0 = c0[T - 1:T, :]
    cnt = cnt0 + c1[T - 1:T, :]
    cnt_ref[...] = cnt
    pg = ((cnt + TM - 1) // TM) * TM
    starts = pg
    d = 1
    while d < N_EXP:
        starts = starts + jnp.concatenate(
            [jnp.zeros((1, d), starts.dtype), starts[:, :-d]], axis=1)
        d *= 2
    starts = starts - pg
    p0 = jnp.sum(oh0 * (starts + c0 - 1), axis=1, keepdims=True)
    p1 = jnp.sum(oh1 * (starts + cnt0 + c1 - 1), axis=1, keepdims=True)
    p0_ref[...] = p0
    p1_ref[...] = p1


def _routing(router_logits):
    p0, p1, wa, wb, cnt = pl.pallas_call(
        _routing_body,
        out_shape=[
            jax.ShapeDtypeStruct((T, 1), jnp.int32),
            jax.ShapeDtypeStruct((T, 1), jnp.int32),
            jax.ShapeDtypeStruct((T, N_EXP), jnp.float32),
            jax.ShapeDtypeStruct((T, N_EXP), jnp.float32),
            jax.ShapeDtypeStruct((1, N_EXP), jnp.int32),
        ],
    )(router_logits)
    cnt = cnt[0]
    pg = ((cnt + TM - 1) // TM) * TM
    starts = jnp.cumsum(pg) - pg
    tile_start = jnp.arange(G, dtype=jnp.int32) * TM
    eid = jnp.clip(
        jnp.searchsorted(starts.astype(jnp.int32), tile_start, side="right")
        - 1, 0, N_EXP - 1).astype(jnp.int32)
    active = (tile_start < (starts[eid] + cnt[eid])).astype(jnp.int32)
    return wa, wb, p0.reshape(T), p1.reshape(T), eid, active


def _ffn_body(eid_ref, act_ref, x_ref, wg_ref, wu_ref, wd_ref, o_ref):
    i = pl.program_id(0)

    @pl.when(act_ref[i] == 1)
    def _():
        x = x_ref[...].astype(jnp.bfloat16)
        g = jnp.dot(x, wg_ref[0], preferred_element_type=jnp.float32)
        u = jnp.dot(x, wu_ref[0], preferred_element_type=jnp.float32)
        a = (g * jax.lax.logistic(g) * u).astype(jnp.bfloat16)
        o_ref[...] = jnp.dot(a, wd_ref[0], preferred_element_type=jnp.float32)


def _ffn(eid, active, xs, wg, wu, wd):
    grid_spec = pltpu.PrefetchScalarGridSpec(
        num_scalar_prefetch=2,
        grid=(G,),
        in_specs=[
            pl.BlockSpec((TM, H), lambda i, eid, act: (i, 0)),
            pl.BlockSpec((1, H, I), lambda i, eid, act: (eid[i], 0, 0)),
            pl.BlockSpec((1, H, I), lambda i, eid, act: (eid[i], 0, 0)),
            pl.BlockSpec((1, I, H), lambda i, eid, act: (eid[i], 0, 0)),
        ],
        out_specs=pl.BlockSpec((TM, H), lambda i, eid, act: (i, 0)),
    )
    return pl.pallas_call(
        _ffn_body,
        grid_spec=grid_spec,
        out_shape=jax.ShapeDtypeStruct((P, H), jnp.float32),
    )(eid, active, xs, wg, wu, wd)


def _combine_body(c0_ref, c1_ref, wa_ref, wb_ref, o_ref):
    o_ref[...] = (c0_ref[...] * wa_ref[:, 0:1]
                  + c1_ref[...] * wb_ref[:, 0:1])


def _combine(c, wa, wb):
    return pl.pallas_call(
        _combine_body,
        grid=(T // TT,),
        in_specs=[
            pl.BlockSpec((TT, H), lambda i: (i, 0)),
            pl.BlockSpec((TT, H), lambda i: (i + T // TT, 0)),
            pl.BlockSpec((TT, N_EXP), lambda i: (i, 0)),
            pl.BlockSpec((TT, N_EXP), lambda i: (i, 0)),
        ],
        out_specs=pl.BlockSpec((TT, H), lambda i: (i, 0)),
        out_shape=jax.ShapeDtypeStruct((T, H), jnp.float32),
    )(c, c, wa, wb)


NC = 2
NS = 16
NW = NC * NS
SL = H // 128

def _sc_mesh():
    return plsc.VectorSubcoreMesh(core_axis_name="c", subcore_axis_name="s")

RB = 16
_DISP_NIT = T // NW // RB
_COMB_NIT = TOPK * T // NW // RB


def _sc_dispatch(hid3, p0_3, p1_3):
    @functools.partial(
        pl.kernel,
        out_type=jax.ShapeDtypeStruct((P, H), jnp.float32),
        mesh=_sc_mesh(),
        scratch_types=[
            pltpu.VMEM((RB, H), jnp.float32),
            pltpu.VMEM((RB, H), jnp.float32),
            pltpu.VMEM((_DISP_NIT, RB), jnp.int32),
            pltpu.VMEM((_DISP_NIT, RB), jnp.int32),
            pltpu.SemaphoreType.DMA,
            pltpu.SemaphoreType.DMA,
            pltpu.SemaphoreType.DMA,
            pltpu.SemaphoreType.DMA,
        ],
    )
    def k(hid_hbm, p0_hbm, p1_hbm, xs_hbm,
          rows0, rows1, i0_v, i1_v, ls0, ls1, ss0, ss1):
        wid = lax.axis_index("s") * NC + lax.axis_index("c")
        base = wid * (T // NW)
        rows = (rows0, rows1)
        lsem = (ls0, ls1)
        ssem = (ss0, ss1)
        pltpu.sync_copy(p0_hbm.at[wid], i0_v)
        pltpu.sync_copy(p1_hbm.at[wid], i1_v)
        ld = [pltpu.async_copy(hid_hbm.at[pl.ds(base, RB)], rows0, ls0), None]
        st = [[], []]
        for j in range(_DISP_NIT):
            b = j & 1
            ld[b].wait()
            if j + 1 < _DISP_NIT:
                for cp in st[1 - b]:
                    cp.wait()
                ld[1 - b] = pltpu.async_copy(
                    hid_hbm.at[pl.ds(base + (j + 1) * RB, RB)],
                    rows[1 - b], lsem[1 - b])
            st[b] = [
                pltpu.async_copy(rows[b], xs_hbm.at[i0_v[j, :]], ssem[b]),
                pltpu.async_copy(rows[b], xs_hbm.at[i1_v[j, :]], ssem[b]),
            ]
        for cp in st[0] + st[1]:
            cp.wait()

    return k(hid3, p0_3, p1_3)


def _sc_combine_gather(d3, p_all_3):
    @functools.partial(
        pl.kernel,
        out_type=jax.ShapeDtypeStruct((TOPK * T, H), jnp.float32),
        mesh=_sc_mesh(),
        scratch_types=[
            pltpu.VMEM((RB, H), jnp.float32),
            pltpu.VMEM((RB, H), jnp.float32),
            pltpu.VMEM((_COMB_NIT, RB), jnp.int32),
            pltpu.SemaphoreType.DMA,
            pltpu.SemaphoreType.DMA,
            pltpu.SemaphoreType.DMA,
            pltpu.SemaphoreType.DMA,
        ],
    )
    def k(d_hbm, p_hbm, c_hbm, rows0, rows1, pidx_v, gs0, gs1, ws0, ws1):
        wid = lax.axis_index("s") * NC + lax.axis_index("c")
        base = wid * (TOPK * T // NW)
        rows = (rows0, rows1)
        gsem = (gs0, gs1)
        wsem = (ws0, ws1)
        pltpu.sync_copy(p_hbm.at[wid], pidx_v)
        g = [pltpu.async_copy(d_hbm.at[pidx_v[0, :]], rows0, gs0), None]
        wr = [None, None]
        for j in range(_COMB_NIT):
            b = j & 1
            g[b].wait()
            if j + 1 < _COMB_NIT:
                if wr[1 - b] is not None:
                    wr[1 - b].wait()
                g[1 - b] = pltpu.async_copy(
                    d_hbm.at[pidx_v[j + 1, :]], rows[1 - b], gsem[1 - b])
            wr[b] = pltpu.async_copy(
                rows[b], c_hbm.at[pl.ds(base + j * RB, RB)], wsem[b])
        wr[0].wait()
        wr[1].wait()

    return k(d3, p_all_3)


def kernel(hidden_states, router_logits, gate_proj, up_proj, down_proj):
    wa, wb, p0, p1, eid, active = _routing(router_logits)
    wg = gate_proj.astype(jnp.bfloat16)
    wu = up_proj.astype(jnp.bfloat16)
    wd = down_proj.astype(jnp.bfloat16)
    xs = _sc_dispatch(hidden_states,
                      p0.reshape(NW, _DISP_NIT, RB),
                      p1.reshape(NW, _DISP_NIT, RB))
    d = _ffn(eid, active, xs, wg, wu, wd)
    p_all = jnp.concatenate([p0, p1]).reshape(NW, _COMB_NIT, RB)
    c = _sc_combine_gather(d, p_all)
    return _combine(c, wa, wb)
